# baseline (device time: 18908 ns/iter reference)
import jax
import jax.numpy as jnp
from jax import lax
from jax.experimental import pallas as pl
from jax.experimental.pallas import tpu as pltpu

N_DEV = 4
B = 2
SQ_LOC = 128
D_MODEL = 512
HQ = 16
H_BLK = 4
DH = 64
SKV = 128
WQ_COLS = 256
WO_ROWS = 256


def kernel(x, Wq, K_ext, V_ext, Wo):
    def body(x_ref, wq_ref, k_ref, v_ref, wo_ref, out_ref,
             x_bf, wq_i8, wo_i8, sc_snd, wq_gath, wo_gath, sc_gath,
             kt_ref, vt_ref, send_sems, recv_sems):
        my_pos = lax.axis_index("i")
        left = (my_pos - 1) % N_DEV
        right = (my_pos + 1) % N_DEV
        opp = (my_pos + 2) % N_DEV

        barrier_sem = pltpu.get_barrier_semaphore()
        for nbr in (opp, left, right):
            pl.semaphore_signal(
                barrier_sem, inc=1,
                device_id=(nbr,), device_id_type=pltpu.DeviceIdType.MESH,
            )

        wq_f = wq_ref[...]
        wo_f = wo_ref[...]
        wq_amax = jnp.maximum(jnp.max(jnp.abs(wq_f)), 1e-20)
        wo_amax = jnp.maximum(jnp.max(jnp.abs(wo_f)), 1e-20)
        wq_i8[...] = jnp.round(wq_f * (127.0 / wq_amax)).astype(jnp.int8)
        wo_i8[...] = jnp.round(wo_f * (127.0 / wo_amax)).astype(jnp.int8)
        sc_snd[0:1, :] = jnp.full((1, 128), wq_amax / 127.0, jnp.float32)
        sc_snd[1:2, :] = jnp.full((1, 128), wo_amax / 127.0, jnp.float32)
        x_bf[...] = x_ref[...].astype(jnp.bfloat16)

        pl.semaphore_wait(barrier_sem, 3)

        def push(src, gath, target, slot, sem):
            return pltpu.make_async_remote_copy(
                src_ref=src,
                dst_ref=gath.at[slot],
                send_sem=send_sems.at[sem],
                recv_sem=recv_sems.at[sem],
                device_id=(target,),
                device_id_type=pltpu.DeviceIdType.MESH,
            )

        sc_to_r = push(sc_snd, sc_gath, right, 0, 0)
        sc_to_l = push(sc_snd, sc_gath, left, 1, 1)
        sc_to_o = push(sc_snd, sc_gath, opp, 2, 2)
        wq_to_r = push(wq_i8, wq_gath, right, 0, 3)
        wq_to_l = push(wq_i8, wq_gath, left, 1, 4)
        wq_to_o = push(wq_i8, wq_gath, opp, 2, 5)
        wo_to_r = push(wo_i8, wo_gath, right, 0, 6)
        wo_to_l = push(wo_i8, wo_gath, left, 1, 7)
        wo_to_o = push(wo_i8, wo_gath, opp, 2, 8)
        for r in (sc_to_r, sc_to_l, sc_to_o,
                  wq_to_r, wo_to_r, wq_to_l, wo_to_l, wq_to_o, wo_to_o):
            r.start()

        for b in range(B):
            for h in range(HQ):
                kt_ref[b, h] = k_ref[b, :, h, :].astype(jnp.bfloat16)
                vt_ref[b, h] = v_ref[b, :, h, :].astype(jnp.bfloat16)

        rows = lax.broadcasted_iota(jnp.int32, (SQ_LOC, SKV), 0)
        cols = lax.broadcasted_iota(jnp.int32, (SQ_LOC, SKV), 1)
        qb = rows // 64 + 2 * my_pos
        kb = cols // 64
        mask = (qb == kb) | (kb == 0) | ((qb + kb) % 3 == 0)
        bias = jnp.where(mask, 0.0, -1e9).astype(jnp.float32)

        def ctx_stage(origin, wq_c, score_scale):
            ctxs_b = []
            for b in range(B):
                qblk = jnp.dot(x_bf[b], wq_c,
                               preferred_element_type=jnp.float32)
                qblk = (qblk * score_scale).astype(jnp.bfloat16)
                parts = []
                for j in range(H_BLK):
                    h = origin * H_BLK + j
                    q = qblk[:, j * DH:(j + 1) * DH]
                    k = kt_ref[b, h]
                    s = lax.dot_general(
                        q, k, (((1,), (1,)), ((), ())),
                        preferred_element_type=jnp.float32,
                    )
                    w = jnp.exp(s + bias)
                    d = jnp.sum(w, axis=1, keepdims=True)
                    c = jnp.dot(w.astype(jnp.bfloat16), vt_ref[b, h],
                                preferred_element_type=jnp.float32) / d
                    parts.append(c)
                ctxs_b.append(
                    jnp.concatenate(parts, axis=1).astype(jnp.bfloat16))
            return ctxs_b

        def out_stage(ctxs_b, wo_c, out_scale=None):
            ps = [jnp.dot(c, wo_c, preferred_element_type=jnp.float32)
                  for c in ctxs_b]
            if out_scale is not None:
                ps = [p * out_scale for p in ps]
            return ps

        acc = out_stage(ctx_stage(my_pos, wq_ref[...].astype(jnp.bfloat16),
                                  0.125),
                        wo_ref[...].astype(jnp.bfloat16))

        def peer_block(origin, slot, sc_rdma, wq_rdma, wo_rdma):
            sc_rdma.wait_recv()
            wq_rdma.wait_recv()
            s_wq = sc_gath[slot, 0, 0]
            s_wo = sc_gath[slot, 1, 0]
            c = ctx_stage(origin, wq_gath[slot].astype(jnp.bfloat16),
                          0.125 * s_wq)
            wo_rdma.wait_recv()
            return out_stage(c, wo_gath[slot].astype(jnp.bfloat16), s_wo)

        p = peer_block(left, 0, sc_to_r, wq_to_r, wo_to_r)
        acc = [a + q for a, q in zip(acc, p)]
        p = peer_block(right, 1, sc_to_l, wq_to_l, wo_to_l)
        acc = [a + q for a, q in zip(acc, p)]
        p = peer_block(opp, 2, sc_to_o, wq_to_o, wo_to_o)
        acc = [a + q for a, q in zip(acc, p)]

        for b in range(B):
            out_ref[b] = acc[b]

        for r in (sc_to_r, sc_to_l, sc_to_o, wq_to_r, wq_to_l, wq_to_o,
                  wo_to_r, wo_to_l, wo_to_o):
            r.wait_send()

    return pl.pallas_call(
        body,
        out_shape=jax.ShapeDtypeStruct((B, SQ_LOC, D_MODEL), jnp.float32),
        in_specs=[pl.BlockSpec(memory_space=pltpu.VMEM)] * 5,
        out_specs=pl.BlockSpec(memory_space=pltpu.VMEM),
        scratch_shapes=[
            pltpu.VMEM((B, SQ_LOC, D_MODEL), jnp.bfloat16),
            pltpu.VMEM((D_MODEL, WQ_COLS), jnp.int8),
            pltpu.VMEM((WO_ROWS, D_MODEL), jnp.int8),
            pltpu.VMEM((2, 128), jnp.float32),
            pltpu.VMEM((3, D_MODEL, WQ_COLS), jnp.int8),
            pltpu.VMEM((3, WO_ROWS, D_MODEL), jnp.int8),
            pltpu.VMEM((3, 2, 128), jnp.float32),
            pltpu.VMEM((B, HQ, SKV, DH), jnp.bfloat16),
            pltpu.VMEM((B, HQ, SKV, DH), jnp.bfloat16),
            pltpu.SemaphoreType.DMA((9,)),
            pltpu.SemaphoreType.DMA((9,)),
        ],
        compiler_params=pltpu.CompilerParams(collective_id=0),
    )(x, Wq, K_ext, V_ext, Wo)


# device time: 13138 ns/iter; 1.4392x vs baseline; 1.4392x over previous
import jax
import jax.numpy as jnp
from jax import lax
from jax.experimental import pallas as pl
from jax.experimental.pallas import tpu as pltpu

N_DEV = 4
B = 2
SQ_LOC = 128
D_MODEL = 512
HQ = 16
H_BLK = 4
DH = 64
SKV = 128
WQ_COLS = 256
WO_ROWS = 256


def kernel(x, Wq, K_ext, V_ext, Wo):
    def body(x_ref, wq_ref, k_ref, v_ref, wo_ref, out_ref,
             x_vm, wq_vm, wo_vm, kt_vm, vt_vm,
             x_bf, wq_i8, wo_i8, sc_snd, wq_gath, wo_gath, sc_gath,
             in_sems, send_sems, recv_sems):
        my_pos = lax.axis_index("i")
        left = (my_pos - 1) % N_DEV
        right = (my_pos + 1) % N_DEV
        opp = (my_pos + 2) % N_DEV

        barrier_sem = pltpu.get_barrier_semaphore()
        for nbr in (opp, left, right):
            pl.semaphore_signal(
                barrier_sem, inc=1,
                device_id=(nbr,), device_id_type=pltpu.DeviceIdType.MESH,
            )

        cp_x = pltpu.make_async_copy(x_ref, x_vm, in_sems.at[0])
        cp_wq = pltpu.make_async_copy(wq_ref, wq_vm, in_sems.at[1])
        cp_wo = pltpu.make_async_copy(wo_ref, wo_vm, in_sems.at[2])
        cp_k = pltpu.make_async_copy(k_ref, kt_vm, in_sems.at[3])
        cp_v = pltpu.make_async_copy(v_ref, vt_vm, in_sems.at[4])
        for c in (cp_wq, cp_wo, cp_x, cp_k, cp_v):
            c.start()

        cp_wq.wait()
        cp_wo.wait()
        wq_f = wq_vm[...]
        wo_f = wo_vm[...]
        wq_amax = jnp.maximum(jnp.max(jnp.abs(wq_f)), 1e-20)
        wo_amax = jnp.maximum(jnp.max(jnp.abs(wo_f)), 1e-20)
        wq_i8[...] = jnp.round(wq_f * (127.0 / wq_amax)).astype(jnp.int8)
        wo_i8[...] = jnp.round(wo_f * (127.0 / wo_amax)).astype(jnp.int8)
        sc_snd[0:1, :] = jnp.full((1, 128), wq_amax / 127.0, jnp.float32)
        sc_snd[1:2, :] = jnp.full((1, 128), wo_amax / 127.0, jnp.float32)
        cp_x.wait()
        x_bf[...] = x_vm[...].astype(jnp.bfloat16)

        pl.semaphore_wait(barrier_sem, 3)

        def push(src, gath, target, slot, sem):
            return pltpu.make_async_remote_copy(
                src_ref=src,
                dst_ref=gath.at[slot],
                send_sem=send_sems.at[sem],
                recv_sem=recv_sems.at[sem],
                device_id=(target,),
                device_id_type=pltpu.DeviceIdType.MESH,
            )

        sc_to_r = push(sc_snd, sc_gath, right, 0, 0)
        sc_to_l = push(sc_snd, sc_gath, left, 1, 1)
        sc_to_o = push(sc_snd, sc_gath, opp, 2, 2)
        wq_to_r = push(wq_i8, wq_gath, right, 0, 3)
        wq_to_l = push(wq_i8, wq_gath, left, 1, 4)
        wq_to_o = push(wq_i8, wq_gath, opp, 2, 5)
        wo_to_r = push(wo_i8, wo_gath, right, 0, 6)
        wo_to_l = push(wo_i8, wo_gath, left, 1, 7)
        wo_to_o = push(wo_i8, wo_gath, opp, 2, 8)
        for r in (sc_to_r, sc_to_l, sc_to_o,
                  wq_to_r, wo_to_r, wq_to_l, wo_to_l, wq_to_o, wo_to_o):
            r.start()


        rows = lax.broadcasted_iota(jnp.int32, (SQ_LOC, SKV), 0)
        cols = lax.broadcasted_iota(jnp.int32, (SQ_LOC, SKV), 1)
        qb = rows // 64 + 2 * my_pos
        kb = cols // 64
        mask = (qb == kb) | (kb == 0) | ((qb + kb) % 3 == 0)
        bias = jnp.where(mask, 0.0, -1e9).astype(jnp.float32)

        def ctx_stage(origin, wq_c, score_scale):
            ctxs_b = []
            for b in range(B):
                qblk = jnp.dot(x_bf[b], wq_c,
                               preferred_element_type=jnp.float32)
                qblk = qblk * score_scale
                parts = []
                for j in range(H_BLK):
                    h = origin * H_BLK + j
                    q = qblk[:, j * DH:(j + 1) * DH]
                    k = kt_vm[b, h]
                    s = lax.dot_general(
                        q, k, (((1,), (0,)), ((), ())),
                        preferred_element_type=jnp.float32,
                    )
                    w = jnp.exp(s + bias)
                    d = jnp.sum(w, axis=1, keepdims=True)
                    c = lax.dot_general(
                        w, vt_vm[b, h], (((1,), (1,)), ((), ())),
                        preferred_element_type=jnp.float32,
                    ) / d
                    parts.append(c)
                ctxs_b.append(
                    jnp.concatenate(parts, axis=1).astype(jnp.bfloat16))
            return ctxs_b

        def out_stage(ctxs_b, wo_c, out_scale=None):
            ps = [jnp.dot(c, wo_c, preferred_element_type=jnp.float32)
                  for c in ctxs_b]
            if out_scale is not None:
                ps = [p * out_scale for p in ps]
            return ps

        cp_k.wait()
        cp_v.wait()
        acc = out_stage(ctx_stage(my_pos, wq_f.astype(jnp.bfloat16),
                                  0.125),
                        wo_f.astype(jnp.bfloat16))

        def peer_block(origin, slot, sc_rdma, wq_rdma, wo_rdma):
            sc_rdma.wait_recv()
            wq_rdma.wait_recv()
            s_wq = sc_gath[slot, 0, 0]
            s_wo = sc_gath[slot, 1, 0]
            c = ctx_stage(origin, wq_gath[slot].astype(jnp.bfloat16),
                          0.125 * s_wq)
            wo_rdma.wait_recv()
            return out_stage(c, wo_gath[slot].astype(jnp.bfloat16), s_wo)

        p = peer_block(left, 0, sc_to_r, wq_to_r, wo_to_r)
        acc = [a + q for a, q in zip(acc, p)]
        p = peer_block(right, 1, sc_to_l, wq_to_l, wo_to_l)
        acc = [a + q for a, q in zip(acc, p)]
        p = peer_block(opp, 2, sc_to_o, wq_to_o, wo_to_o)
        acc = [a + q for a, q in zip(acc, p)]

        for b in range(B):
            out_ref[b] = acc[b]

        for r in (sc_to_r, sc_to_l, sc_to_o, wq_to_r, wq_to_l, wq_to_o,
                  wo_to_r, wo_to_l, wo_to_o):
            r.wait_send()

    return pl.pallas_call(
        body,
        out_shape=jax.ShapeDtypeStruct((B, SQ_LOC, D_MODEL), jnp.float32),
        in_specs=[pl.BlockSpec(memory_space=pltpu.MemorySpace.HBM)] * 5,
        out_specs=pl.BlockSpec(memory_space=pltpu.VMEM),
        scratch_shapes=[
            pltpu.VMEM((B, SQ_LOC, D_MODEL), jnp.float32),
            pltpu.VMEM((D_MODEL, WQ_COLS), jnp.float32),
            pltpu.VMEM((WO_ROWS, D_MODEL), jnp.float32),
            pltpu.VMEM((B, HQ, DH, SKV), jnp.float32),
            pltpu.VMEM((B, HQ, DH, SKV), jnp.float32),
            pltpu.VMEM((B, SQ_LOC, D_MODEL), jnp.bfloat16),
            pltpu.VMEM((D_MODEL, WQ_COLS), jnp.int8),
            pltpu.VMEM((WO_ROWS, D_MODEL), jnp.int8),
            pltpu.VMEM((2, 128), jnp.float32),
            pltpu.VMEM((3, D_MODEL, WQ_COLS), jnp.int8),
            pltpu.VMEM((3, WO_ROWS, D_MODEL), jnp.int8),
            pltpu.VMEM((3, 2, 128), jnp.float32),
            pltpu.SemaphoreType.DMA((5,)),
            pltpu.SemaphoreType.DMA((9,)),
            pltpu.SemaphoreType.DMA((9,)),
        ],
        compiler_params=pltpu.CompilerParams(collective_id=0),
    )(*[pltpu.with_memory_space_constraint(a, pltpu.MemorySpace.HBM)
        for a in (x, Wq, jnp.transpose(K_ext, (0, 2, 3, 1)),
                  jnp.transpose(V_ext, (0, 2, 3, 1)), Wo)])
